# Initial kernel scaffold; baseline (speedup 1.0000x reference)
import jax
import jax.numpy as jnp
from jax.experimental import pallas as pl


def kernel(x_cat1, x_cat2, x_num, tables, W1, b1, W2, b2):
    raise NotImplementedError("write your pallas kernel here")


# speedup vs baseline: 1.2766x; 1.2766x over previous
import functools

import jax
import jax.numpy as jnp
from jax import lax
from jax.experimental import pallas as pl
from jax.experimental.pallas import tpu as pltpu
from jax.experimental.pallas import tpu_sc as plsc

N_FIELDS = 26
VOCAB = 100000
EMB_DIM = 10
HIDDEN = 50
N_CLASSES = 5
N_NUM = 13
BATCH = 16384
CAT_DIM = N_FIELDS * EMB_DIM

_NC = 2
_NS = 16
_NW = _NC * _NS
_R_TOTAL = BATCH * N_FIELDS
_R_PER_W = _R_TOTAL // _NW
_IW = 128
_ROWS_PER_W = _R_PER_W // _IW
_N_CHUNKS = 2
_CHUNK_ROWS = _ROWS_PER_W // _N_CHUNKS
_CHUNK = _CHUNK_ROWS * _IW


@functools.cache
def _get_sc_gather():
    @functools.partial(
        pl.kernel,
        out_type=jax.ShapeDtypeStruct((_R_TOTAL, EMB_DIM), jnp.float32),
        mesh=plsc.VectorSubcoreMesh(
            core_axis_name="c", subcore_axis_name="s",
            num_cores=_NC, num_subcores=_NS,
        ),
        scratch_types=[
            pltpu.VMEM((_ROWS_PER_W, _IW), jnp.int32),
            pltpu.VMEM((_CHUNK, EMB_DIM), jnp.float32),
            pltpu.SemaphoreType.DMA,
        ],
        compiler_params=pltpu.CompilerParams(use_tc_tiling_on_sc=False),
    )
    def _sc_gather(idx_hbm, tab_hbm, out_hbm, idx_v, rows_v, sem):
        wid = lax.axis_index("s") * _NC + lax.axis_index("c")
        row0 = wid * _ROWS_PER_W
        pltpu.sync_copy(idx_hbm.at[pl.ds(row0, _ROWS_PER_W)], idx_v)
        for c in range(_N_CHUNKS):

            def gather_one(j, _):
                r = c * _CHUNK_ROWS + j
                pltpu.async_copy(
                    tab_hbm.at[idx_v.at[r]],
                    rows_v.at[pl.ds(j * _IW, _IW)],
                    sem,
                ).wait()
                return 0

            lax.fori_loop(0, _CHUNK_ROWS, gather_one, 0)
            pltpu.sync_copy(
                rows_v,
                out_hbm.at[pl.ds((row0 + c * _CHUNK_ROWS) * _IW, _CHUNK)],
            )

    return _sc_gather


def _mlp_body(xe_ref, xn_ref, w1a_ref, w1b_ref, b1_ref, w2_ref, b2_ref, o_ref):
    h = jnp.dot(xe_ref[...], w1a_ref[...], preferred_element_type=jnp.float32)
    h = h + jnp.dot(xn_ref[...], w1b_ref[...], preferred_element_type=jnp.float32)
    h = jnp.maximum(h + b1_ref[...], 0.0)
    l = jnp.dot(h, w2_ref[...], preferred_element_type=jnp.float32) + b2_ref[...]
    m = jnp.max(l, axis=1, keepdims=True)
    e = jnp.exp(l - m)
    o_ref[...] = e / jnp.sum(e, axis=1, keepdims=True)


_BB = 2048


def _mlp(x_emb, x_num, w1a, w1b, b1, w2, b2):
    grid = (BATCH // _BB,)
    return pl.pallas_call(
        _mlp_body,
        grid=grid,
        in_specs=[
            pl.BlockSpec((_BB, CAT_DIM), lambda i: (i, 0)),
            pl.BlockSpec((_BB, N_NUM), lambda i: (i, 0)),
            pl.BlockSpec((CAT_DIM, HIDDEN), lambda i: (0, 0)),
            pl.BlockSpec((N_NUM, HIDDEN), lambda i: (0, 0)),
            pl.BlockSpec((1, HIDDEN), lambda i: (0, 0)),
            pl.BlockSpec((HIDDEN, N_CLASSES), lambda i: (0, 0)),
            pl.BlockSpec((1, N_CLASSES), lambda i: (0, 0)),
        ],
        out_specs=pl.BlockSpec((_BB, N_CLASSES), lambda i: (i, 0)),
        out_shape=jax.ShapeDtypeStruct((BATCH, N_CLASSES), jnp.float32),
    )(x_emb, x_num, w1a, w1b, b1, w2, b2)


def kernel(x_cat1, x_cat2, x_num, tables, W1, b1, W2, b2):
    idx = jnp.concatenate(
        [x_cat1.astype(jnp.int32), x_cat2.astype(jnp.int32)], axis=1
    )
    idx = idx + (jnp.arange(N_FIELDS, dtype=jnp.int32) * VOCAB)[None, :]
    idx_rows = idx.reshape(_R_TOTAL // _IW, _IW)
    tab = tables.reshape(N_FIELDS * VOCAB, EMB_DIM)
    x_emb = _get_sc_gather()(idx_rows, tab).reshape(BATCH, CAT_DIM)
    return _mlp(
        x_emb,
        x_num,
        W1[:CAT_DIM],
        W1[CAT_DIM:],
        b1.reshape(1, HIDDEN),
        W2,
        b2.reshape(1, N_CLASSES),
    )
